# baseline (speedup 1.0000x reference)
import functools

import jax
import jax.numpy as jnp
from jax import lax
from jax.experimental import pallas as pl
from jax.experimental.pallas import tpu as pltpu
from jax.experimental.pallas import tpu_sc as plsc

N = 10000
E = 320000
D = 128
H = 64
G = 64
C = 2
NC, NS, L = 2, 16, 16
NW = NC * NS
NP = 10112
RPT = NP // NS
CH = 128
EPT = E // NW
K = 81
NBUF = 3
KO = K // NBUF
SLOTS = K * CH
HC = H // NC
EPT2 = E // NS
K2 = 160
NBUF2 = 8
KO2 = K2 // NBUF2
SLOTS2 = K2 * CH
PAD_SRC = N
PAD_DST = N + 8
GP = 72
BPT = RPT
NBP = NP - N

_MESH = plsc.VectorSubcoreMesh(
    core_axis_name="c", subcore_axis_name="s", num_cores=NC, num_subcores=NS)
_SC_PARAMS = pltpu.CompilerParams(use_tc_tiling_on_sc=False)
_SC_PARAMS_NL = pltpu.CompilerParams(use_tc_tiling_on_sc=False,
                                     needs_layout_passes=False)


def _pad_tail(idx, start, count, value):
    for i in range(count // 16):
        idx[pl.ds(start + 16 * i, 16)] = jnp.full((16,), value, jnp.int32)


@functools.partial(
    pl.kernel,
    out_type=(jax.ShapeDtypeStruct((NC, NP), jnp.float32),
              jax.ShapeDtypeStruct((NC, GP, L), jnp.float32)),
    mesh=_MESH,
    scratch_types=[
        pltpu.VMEM_SHARED((NP, L), jnp.float32),
        pltpu.VMEM_SHARED((GP, L), jnp.float32),
        pltpu.VMEM((SLOTS,), jnp.int32),
        pltpu.VMEM((5 * CH,), jnp.int32),
        pltpu.VMEM((CH, L), jnp.float32),
        pltpu.VMEM((RPT + 8, L), jnp.float32),
        pltpu.VMEM((RPT + 8,), jnp.float32),
        pltpu.SemaphoreType.DMA((NBUF,)),
    ],
    compiler_params=_SC_PARAMS_NL,
)
def _sc_deg(ei_hbm, bidx_hbm, z_hbm, ones_hbm, deg_hbm, cnt_hbm,
            acc, bacc, didx, bidx, ones_v, vb16, dcomp, ssem):
    c = lax.axis_index("c")
    s = lax.axis_index("s")
    wid = s * NC + c
    base = s * RPT
    pltpu.sync_copy(z_hbm.at[pl.ds(base, RPT)], acc.at[pl.ds(base, RPT)])

    @pl.when(s == 0)
    def _():
        pltpu.sync_copy(z_hbm.at[pl.ds(0, GP)], bacc)

    _pad_tail(didx, EPT, SLOTS - EPT, PAD_DST)
    pltpu.sync_copy(ei_hbm.at[1, pl.ds(wid * EPT, EPT)], didx.at[pl.ds(0, EPT)])
    pltpu.sync_copy(ones_hbm, ones_v)

    @pl.when(c == 0)
    def _():
        _pad_tail(bidx, 4 * CH, CH, G)
        pltpu.sync_copy(bidx_hbm.at[pl.ds(s * BPT, BPT)],
                        bidx.at[pl.ds(0, BPT)])
    plsc.subcore_barrier()

    for b in range(NBUF):
        pltpu.async_copy(ones_v, acc.at[didx.at[pl.ds(b * CH, CH)]],
                         ssem.at[b], add=True)

    def body(j0, carry):
        for b in range(NBUF):
            j = j0 * NBUF + b
            pltpu.make_async_copy(ones_v, acc.at[didx.at[pl.ds(j * CH, CH)]],
                                  ssem.at[b]).wait()

            @pl.when(j0 < KO - 1)
            def _():
                pltpu.async_copy(ones_v,
                                 acc.at[didx.at[pl.ds((j + NBUF) * CH, CH)]],
                                 ssem.at[b], add=True)
        return carry

    lax.fori_loop(0, KO, body, 0)

    @pl.when(c == 0)
    def _():
        for j in range(5):
            pltpu.sync_copy(ones_v, bacc.at[bidx.at[pl.ds(j * CH, CH)]],
                            add=True)
    plsc.subcore_barrier()
    pltpu.sync_copy(acc.at[pl.ds(base, RPT)], vb16.at[pl.ds(0, RPT)])
    zero16 = jnp.zeros((L,), jnp.int32)
    iota16 = lax.iota(jnp.int32, L)

    def compact(g, carry):
        rowv = iota16 + g * L
        dcomp[pl.ds(g * L, L)] = plsc.load_gather(vb16, [rowv, zero16])
        return carry

    lax.fori_loop(0, (RPT + 8) // L, compact, 0)
    pltpu.sync_copy(dcomp.at[pl.ds(0, RPT)], deg_hbm.at[c].at[pl.ds(base, RPT)])

    @pl.when(s == 0)
    def _():
        pltpu.sync_copy(bacc, cnt_hbm.at[c])


@functools.partial(
    pl.kernel,
    out_type=jax.ShapeDtypeStruct((NP, H), jnp.float32),
    mesh=_MESH,
    scratch_types=[
        pltpu.VMEM_SHARED((NP, HC), jnp.float32),
        pltpu.VMEM_SHARED((NP, HC), jnp.float32),
        pltpu.VMEM((SLOTS2,), jnp.int32),
        pltpu.VMEM((SLOTS2,), jnp.int32),
        pltpu.VMEM((NBUF2, CH, HC), jnp.float32),
        pltpu.SemaphoreType.DMA((NBUF2,)),
        pltpu.SemaphoreType.DMA((NBUF2,)),
    ],
    compiler_params=_SC_PARAMS,
)
def _sc_agg(y_hbm, ei_hbm, out_hbm, acc, ybuf, sidx, didx, rows, gsem, ssem):
    c = lax.axis_index("c")
    s = lax.axis_index("s")
    base = s * RPT

    pltpu.sync_copy(y_hbm.at[pl.ds(base, RPT), pl.ds(c * HC, HC)],
                    ybuf.at[pl.ds(base, RPT)])
    pltpu.sync_copy(y_hbm.at[pl.ds(base, RPT), pl.ds(c * HC, HC)],
                    acc.at[pl.ds(base, RPT)])

    _pad_tail(sidx, EPT2, SLOTS2 - EPT2, PAD_SRC)
    _pad_tail(didx, EPT2, SLOTS2 - EPT2, PAD_DST)
    pltpu.sync_copy(ei_hbm.at[0, pl.ds(s * EPT2, EPT2)], sidx.at[pl.ds(0, EPT2)])
    pltpu.sync_copy(ei_hbm.at[1, pl.ds(s * EPT2, EPT2)], didx.at[pl.ds(0, EPT2)])
    plsc.subcore_barrier()

    for b in range(NBUF2):
        pltpu.async_copy(ybuf.at[sidx.at[pl.ds(b * CH, CH)]], rows.at[b],
                         gsem.at[b])

    def body(j0, carry):
        for b in range(NBUF2):
            j = j0 * NBUF2 + b
            pltpu.make_async_copy(ybuf.at[sidx.at[pl.ds(j * CH, CH)]],
                                  rows.at[b], gsem.at[b]).wait()
            pltpu.async_copy(rows.at[b], acc.at[didx.at[pl.ds(j * CH, CH)]],
                             ssem.at[b], add=True)

            @pl.when(j0 < KO2 - 1)
            def _():
                pltpu.make_async_copy(rows.at[b],
                                      acc.at[didx.at[pl.ds(j * CH, CH)]],
                                      ssem.at[b]).wait()
                pltpu.async_copy(ybuf.at[sidx.at[pl.ds((j + NBUF2) * CH, CH)]],
                                 rows.at[b], gsem.at[b])
        return carry

    lax.fori_loop(0, KO2, body, 0)
    for b in range(NBUF2):
        pltpu.make_async_copy(
            rows.at[b], acc.at[didx.at[pl.ds((K2 - NBUF2 + b) * CH, CH)]],
            ssem.at[b]).wait()
    plsc.subcore_barrier()
    pltpu.sync_copy(acc.at[pl.ds(base, RPT)],
                    out_hbm.at[pl.ds(base, RPT), pl.ds(c * HC, HC)])


def _tc_xw_body(x_ref, w_ref, xw_ref):
    xw = jnp.dot(x_ref[...], w_ref[...], preferred_element_type=jnp.float32)
    xw_ref[...] = jnp.pad(xw, ((0, NP - N), (0, 0)))


_tc_xw = pl.pallas_call(
    _tc_xw_body,
    out_shape=jax.ShapeDtypeStruct((NP, H), jnp.float32),
)


def _tc_scale_body(xw_ref, degp_ref, y_ref, dinv_ref):
    deg = (degp_ref[0, :] + degp_ref[1, :]).reshape(NP, 1) + 1.0
    dinv = lax.rsqrt(deg)
    dinv_ref[...] = jnp.broadcast_to(dinv, (NP, L))
    y_ref[...] = xw_ref[...] * dinv


_tc_scale = pl.pallas_call(
    _tc_scale_body,
    out_shape=(jax.ShapeDtypeStruct((NP, H), jnp.float32),
               jax.ShapeDtypeStruct((NP, L), jnp.float32)),
)


def _tc_mid_body(p_ref, dinv_ref, b1_ref, w2_ref, y2_ref):
    dinv = dinv_ref[:, 0:1]
    h1 = jnp.maximum(p_ref[...] * dinv + b1_ref[...], 0.0)
    y2 = jnp.dot(h1, w2_ref[...], preferred_element_type=jnp.float32) * dinv
    row = lax.broadcasted_iota(jnp.int32, (NP, 1), 0)
    y2_ref[...] = jnp.where(row < N, y2, 0.0)


_tc_mid = pl.pallas_call(
    _tc_mid_body,
    out_shape=jax.ShapeDtypeStruct((NP, H), jnp.float32),
)


def _tc_final_body(q_ref, dinv_ref, b2_ref, cnt_ref, l1w_ref, l1b_ref,
                   l2w_ref, l2b_ref, out_ref):
    dinv = dinv_ref[:, 0:1]
    h2 = jnp.maximum(q_ref[...] * dinv + b2_ref[...], 0.0)
    crow = cnt_ref[0:1, 0:G, 0] + cnt_ref[1:2, 0:G, 0]
    ccol = cnt_ref[0, 0:G, 0:1] + cnt_ref[1, 0:G, 0:1]
    tri = (lax.broadcasted_iota(jnp.int32, (G, G), 0)
           <= lax.broadcasted_iota(jnp.int32, (G, G), 1)).astype(jnp.float32)
    cum = jnp.dot(crow, tri, preferred_element_type=jnp.float32)
    start = cum - crow
    rowi = lax.broadcasted_iota(jnp.int32, (NP, 1), 0).astype(jnp.float32)
    onehot = ((rowi >= start) & (rowi < cum)).astype(jnp.float32)
    dn = (((0,), (0,)), ((), ()))
    psum = lax.dot_general(onehot, h2, dn, preferred_element_type=jnp.float32)
    pooled = psum / jnp.maximum(ccol, 1.0)
    z = jnp.maximum(
        jnp.dot(pooled, l1w_ref[...], preferred_element_type=jnp.float32)
        + l1b_ref[...], 0.0)
    logits = jnp.dot(z, l2w_ref[...], preferred_element_type=jnp.float32) \
        + l2b_ref[...]
    m = jnp.max(logits, axis=1, keepdims=True)
    e = jnp.exp(logits - m)
    out_ref[...] = (logits - m) - jnp.log(jnp.sum(e, axis=1, keepdims=True))


_tc_final = pl.pallas_call(
    _tc_final_body,
    out_shape=jax.ShapeDtypeStruct((G, C), jnp.float32),
)


@jax.jit
def kernel(x, edge_index, batch_index, W1, b1, W2, b2,
           lin1_w, lin1_b, lin2_w, lin2_b):
    i32, f32 = jnp.int32, jnp.float32
    ei = edge_index.astype(i32)
    bidx = jnp.concatenate(
        [batch_index.astype(i32), jnp.full((NBP,), G, i32)])
    z16 = jnp.zeros((NP, L), f32)
    ones_rows = jnp.ones((CH, L), f32)

    xw = _tc_xw(x, W1)
    degp, cntp = _sc_deg(ei, bidx, z16, ones_rows)
    y1, dinv16 = _tc_scale(xw, degp)
    p = _sc_agg(y1, ei)
    y2 = _tc_mid(p, dinv16, b1.reshape(1, H), W2)
    q = _sc_agg(y2, ei)
    return _tc_final(q, dinv16, b2.reshape(1, H), cntp,
                     lin1_w, lin1_b.reshape(1, H // 2),
                     lin2_w, lin2_b.reshape(1, C))
